# baseline (speedup 1.0000x reference)
import functools

import jax
import jax.numpy as jnp
from jax import lax
from jax.experimental import pallas as pl
from jax.experimental.pallas import tpu as pltpu
from jax.experimental.pallas import tpu_sc as plsc

_TAU_WORD = 0.1
_ALPHA = 0.7

_NC = 2
_NS = 16
_L = 16
_NW = _NC * _NS

_GDN = lax.GatherDimensionNumbers(
    offset_dims=(), collapsed_slice_dims=(0,), start_index_map=(0,))


def _permute(x, idx):
    return lax.gather(x, idx[:, None], _GDN, (1,),
                      mode=lax.GatherScatterMode.PROMISE_IN_BOUNDS)


def _lane_total(x):
    iota = lax.iota(jnp.int32, _L)
    for shift in (8, 4, 2, 1):
        x = x + _permute(x, iota ^ shift)
    return x


def _make_sc_kernel(N, V):
    RW = N // _NW
    C = 4
    NCHUNK = RW // C
    VCH = V // _L
    mesh = plsc.VectorSubcoreMesh(core_axis_name="c", subcore_axis_name="s",
                                  num_cores=_NC, num_subcores=_NS)

    @functools.partial(
        pl.kernel,
        mesh=mesh,
        compiler_params=pltpu.CompilerParams(needs_layout_passes=False),
        out_type=jax.ShapeDtypeStruct((_NW, 3, _L), jnp.float32),
        scratch_types=[
            pltpu.VMEM((RW,), jnp.int32),
            pltpu.VMEM((NCHUNK, C), jnp.int32),
            pltpu.VMEM((RW,), jnp.int32),
            pltpu.VMEM((RW,), jnp.float32),
            pltpu.VMEM((RW,), jnp.float32),
            pltpu.VMEM((2, C, V), jnp.float32),
            pltpu.VMEM((2, C, V), jnp.float32),
            pltpu.VMEM((3, _L), jnp.float32),
            pltpu.SemaphoreType.DMA,
            pltpu.SemaphoreType.DMA,
            pltpu.SemaphoreType.DMA,
        ],
    )
    def sc_kernel(logp_hbm, logp1d_hbm, sim_hbm, tgt_hbm, tgt3_hbm, mask_hbm,
                  out_hbm, idx_v, idx2_v, fidx_v, mask_v, picked_v, sim_buf,
                  logp_buf, out_stage, sem_sim, sem_logp, sem_misc):
        wid = lax.axis_index("s") * _NC + lax.axis_index("c")
        base = wid * RW

        pltpu.sync_copy(tgt_hbm.at[pl.ds(base, RW)], idx_v)
        pltpu.sync_copy(tgt3_hbm.at[wid], idx2_v)
        pltpu.sync_copy(mask_hbm.at[pl.ds(base, RW)], mask_v)

        iota16 = lax.iota(jnp.int32, _L)
        for g in range(RW // _L):
            t16 = idx_v[pl.ds(g * _L, _L)]
            rows = (base + g * _L) + iota16
            fidx_v[pl.ds(g * _L, _L)] = t16 + rows * V
        picked_dma = pltpu.async_copy(logp1d_hbm.at[fidx_v], picked_v,
                                      sem_misc)

        def start_chunk(c, slot):
            sim = pltpu.async_copy(
                sim_hbm.at[idx2_v.at[c]], sim_buf.at[slot], sem_sim)
            lp = pltpu.async_copy(
                logp_hbm.at[pl.ds(base + c * C, C)], logp_buf.at[slot],
                sem_logp)
            return sim, lp

        smooth_acc = jnp.zeros((_L,), jnp.float32)
        inflight = start_chunk(0, 0)
        inv_tau = jnp.float32(1.0 / _TAU_WORD)

        for c in range(NCHUNK):
            slot = c % 2
            inflight[0].wait()
            inflight[1].wait()
            if c + 1 < NCHUNK:
                inflight = start_chunk(c + 1, (c + 1) % 2)
            for r in range(C):
                def body(j, carry):
                    z, d = carry
                    off = j * _L
                    e = jnp.exp(sim_buf[slot, r, pl.ds(off, _L)] * inv_tau)
                    z = z + e
                    d = d + e * logp_buf[slot, r, pl.ds(off, _L)]
                    return z, d
                z, d = lax.fori_loop(
                    0, VCH, body,
                    (jnp.zeros((_L,), jnp.float32),
                     jnp.zeros((_L,), jnp.float32)))
                zt = _lane_total(z)
                dt = _lane_total(d)
                m = plsc.load_gather(
                    mask_v, [jnp.full((_L,), c * C + r, jnp.int32)])
                smooth_acc = smooth_acc + m * (dt / zt)

        picked_dma.wait()
        ml_acc = jnp.zeros((_L,), jnp.float32)
        msum_acc = jnp.zeros((_L,), jnp.float32)
        for g in range(RW // _L):
            mv = mask_v[pl.ds(g * _L, _L)]
            ml_acc = ml_acc + picked_v[pl.ds(g * _L, _L)] * mv
            msum_acc = msum_acc + mv

        out_stage[0, :] = smooth_acc
        out_stage[1, :] = ml_acc
        out_stage[2, :] = msum_acc
        pltpu.sync_copy(out_stage, out_hbm.at[wid])

    return sc_kernel


@jax.jit
def kernel(logp, sim_matrix, target, mask):
    B, T, V = logp.shape
    N = B * T
    flat_logp = logp.reshape(N, V)
    logp1d = logp.reshape(N * V)
    idx = target.reshape(N).astype(jnp.int32)
    flat_mask = mask.reshape(N)

    idx3 = idx.reshape(_NW, -1, 4)
    parts = _make_sc_kernel(N, V)(flat_logp, logp1d, sim_matrix, idx, idx3,
                                  flat_mask)
    smooth_sum = jnp.sum(parts[:, 0, :]) / _L
    ml_sum = jnp.sum(parts[:, 1, :])
    msum = jnp.sum(parts[:, 2, :])
    ml_output = -ml_sum / msum
    output = -smooth_sum / msum
    final = _ALPHA * output + (1.0 - _ALPHA) * ml_output
    return jnp.stack([final, ml_output])


# speedup vs baseline: 1.2274x; 1.2274x over previous
import functools

import jax
import jax.numpy as jnp
from jax import lax
from jax.experimental import pallas as pl
from jax.experimental.pallas import tpu as pltpu
from jax.experimental.pallas import tpu_sc as plsc

_TAU_WORD = 0.1
_ALPHA = 0.7

_NC = 2
_NS = 16
_L = 16
_NW = _NC * _NS

_GDN = lax.GatherDimensionNumbers(
    offset_dims=(), collapsed_slice_dims=(0,), start_index_map=(0,))


def _permute(x, idx):
    return lax.gather(x, idx[:, None], _GDN, (1,),
                      mode=lax.GatherScatterMode.PROMISE_IN_BOUNDS)


def _lane_total(x):
    iota = lax.iota(jnp.int32, _L)
    for shift in (8, 4, 2, 1):
        x = x + _permute(x, iota ^ shift)
    return x


def _make_sc_kernel(N, V):
    RW = N // _NW
    C = 4
    NCHUNK = RW // C
    VCH = V // _L
    mesh = plsc.VectorSubcoreMesh(core_axis_name="c", subcore_axis_name="s",
                                  num_cores=_NC, num_subcores=_NS)

    @functools.partial(
        pl.kernel,
        mesh=mesh,
        compiler_params=pltpu.CompilerParams(needs_layout_passes=False),
        out_type=jax.ShapeDtypeStruct((_NW, 3, _L), jnp.float32),
        scratch_types=[
            pltpu.VMEM((RW,), jnp.int32),
            pltpu.VMEM((NCHUNK, C), jnp.int32),
            pltpu.VMEM((RW,), jnp.int32),
            pltpu.VMEM((RW,), jnp.float32),
            pltpu.VMEM((RW,), jnp.float32),
            pltpu.VMEM((2, C, V), jnp.float32),
            pltpu.VMEM((2, C, V), jnp.float32),
            pltpu.VMEM((3, _L), jnp.float32),
            pltpu.SemaphoreType.DMA,
            pltpu.SemaphoreType.DMA,
            pltpu.SemaphoreType.DMA,
        ],
    )
    def sc_kernel(logp_hbm, logp1d_hbm, sim_hbm, tgt_hbm, tgt3_hbm, mask_hbm,
                  out_hbm, idx_v, idx2_v, fidx_v, mask_v, picked_v, sim_buf,
                  logp_buf, out_stage, sem_sim, sem_logp, sem_misc):
        wid = lax.axis_index("s") * _NC + lax.axis_index("c")
        base = wid * RW

        pltpu.sync_copy(tgt_hbm.at[pl.ds(base, RW)], idx_v)
        pltpu.sync_copy(tgt3_hbm.at[wid], idx2_v)
        pltpu.sync_copy(mask_hbm.at[pl.ds(base, RW)], mask_v)

        iota16 = lax.iota(jnp.int32, _L)
        for g in range(RW // _L):
            t16 = idx_v[pl.ds(g * _L, _L)]
            rows = (base + g * _L) + iota16
            fidx_v[pl.ds(g * _L, _L)] = t16 + rows * V
        picked_dma = pltpu.async_copy(logp1d_hbm.at[fidx_v], picked_v,
                                      sem_misc)

        def start_chunk(c, slot):
            sim = pltpu.async_copy(
                sim_hbm.at[idx2_v.at[c]], sim_buf.at[slot], sem_sim)
            lp = pltpu.async_copy(
                logp_hbm.at[pl.ds(base + c * C, C)], logp_buf.at[slot],
                sem_logp)
            return sim, lp

        smooth_acc = jnp.zeros((_L,), jnp.float32)
        inflight = start_chunk(0, 0)
        inv_tau = jnp.float32(1.0 / _TAU_WORD)

        for c in range(NCHUNK):
            slot = c % 2
            inflight[0].wait()
            inflight[1].wait()
            if c + 1 < NCHUNK:
                inflight = start_chunk(c + 1, (c + 1) % 2)
            for r in range(C):
                UNROLL = 8
                NACC = 4

                def body(j, carry):
                    zs = list(carry[:NACC])
                    ds_ = list(carry[NACC:])
                    base_off = j * (_L * UNROLL)
                    for u in range(UNROLL):
                        off = base_off + u * _L
                        s = sim_buf[slot, r, pl.ds(off, _L)]
                        lg = logp_buf[slot, r, pl.ds(off, _L)]
                        e = jnp.exp(s * inv_tau)
                        a = u % NACC
                        zs[a] = zs[a] + e
                        ds_[a] = ds_[a] + e * lg
                    return tuple(zs) + tuple(ds_)

                zero = jnp.zeros((_L,), jnp.float32)
                acc = lax.fori_loop(0, VCH // UNROLL, body, (zero,) * (2 * NACC))
                z = (acc[0] + acc[1]) + (acc[2] + acc[3])
                d = (acc[4] + acc[5]) + (acc[6] + acc[7])
                zt = _lane_total(z)
                dt = _lane_total(d)
                m = plsc.load_gather(
                    mask_v, [jnp.full((_L,), c * C + r, jnp.int32)])
                smooth_acc = smooth_acc + m * (dt / zt)

        picked_dma.wait()
        ml_acc = jnp.zeros((_L,), jnp.float32)
        msum_acc = jnp.zeros((_L,), jnp.float32)
        for g in range(RW // _L):
            mv = mask_v[pl.ds(g * _L, _L)]
            ml_acc = ml_acc + picked_v[pl.ds(g * _L, _L)] * mv
            msum_acc = msum_acc + mv

        out_stage[0, :] = smooth_acc
        out_stage[1, :] = ml_acc
        out_stage[2, :] = msum_acc
        pltpu.sync_copy(out_stage, out_hbm.at[wid])

    return sc_kernel


@jax.jit
def kernel(logp, sim_matrix, target, mask):
    B, T, V = logp.shape
    N = B * T
    flat_logp = logp.reshape(N, V)
    logp1d = logp.reshape(N * V)
    idx = target.reshape(N).astype(jnp.int32)
    flat_mask = mask.reshape(N)

    idx3 = idx.reshape(_NW, -1, 4)
    parts = _make_sc_kernel(N, V)(flat_logp, logp1d, sim_matrix, idx, idx3,
                                  flat_mask)
    smooth_sum = jnp.sum(parts[:, 0, :]) / _L
    ml_sum = jnp.sum(parts[:, 1, :])
    msum = jnp.sum(parts[:, 2, :])
    ml_output = -ml_sum / msum
    output = -smooth_sum / msum
    final = _ALPHA * output + (1.0 - _ALPHA) * ml_output
    return jnp.stack([final, ml_output])


# speedup vs baseline: 1.7076x; 1.3912x over previous
import functools

import jax
import jax.numpy as jnp
from jax import lax
from jax.experimental import pallas as pl
from jax.experimental.pallas import tpu as pltpu
from jax.experimental.pallas import tpu_sc as plsc

_TAU_WORD = 0.1
_ALPHA = 0.7

_NC = 2
_NS = 16
_L = 16
_NW = _NC * _NS

_GDN = lax.GatherDimensionNumbers(
    offset_dims=(), collapsed_slice_dims=(0,), start_index_map=(0,))


def _permute(x, idx):
    return lax.gather(x, idx[:, None], _GDN, (1,),
                      mode=lax.GatherScatterMode.PROMISE_IN_BOUNDS)


def _lane_total(x):
    iota = lax.iota(jnp.int32, _L)
    for shift in (8, 4, 2, 1):
        x = x + _permute(x, iota ^ shift)
    return x


def _make_sc_kernel(N, V):
    RW = N // _NW
    C = 4
    NCHUNK = RW // C
    VCH = V // _L
    mesh = plsc.VectorSubcoreMesh(core_axis_name="c", subcore_axis_name="s",
                                  num_cores=_NC, num_subcores=_NS)

    @functools.partial(
        pl.kernel,
        mesh=mesh,
        compiler_params=pltpu.CompilerParams(needs_layout_passes=False),
        out_type=jax.ShapeDtypeStruct((_NW, 3, _L), jnp.float32),
        scratch_types=[
            pltpu.VMEM((RW,), jnp.int32),
            pltpu.VMEM((NCHUNK, C), jnp.int32),
            pltpu.VMEM((RW,), jnp.float32),
            pltpu.VMEM((2, C, V), jnp.float32),
            pltpu.VMEM((2, C, V), jnp.float32),
            pltpu.VMEM((3, _L), jnp.float32),
            pltpu.SemaphoreType.DMA,
            pltpu.SemaphoreType.DMA,
        ],
    )
    def sc_kernel(logp_hbm, sim_hbm, tgt_hbm, tgt3_hbm, mask_hbm,
                  out_hbm, idx_v, idx2_v, mask_v, sim_buf,
                  logp_buf, out_stage, sem_sim, sem_logp):
        wid = lax.axis_index("s") * _NC + lax.axis_index("c")
        base = wid * RW

        pltpu.sync_copy(tgt_hbm.at[pl.ds(base, RW)], idx_v)
        pltpu.sync_copy(tgt3_hbm.at[wid], idx2_v)
        pltpu.sync_copy(mask_hbm.at[pl.ds(base, RW)], mask_v)

        iota16 = lax.iota(jnp.int32, _L)
        rowsel = iota16 % C
        firstcopy = (iota16 < C).astype(jnp.float32)

        def start_chunk(c, slot):
            sim = pltpu.async_copy(
                sim_hbm.at[idx2_v.at[c]], sim_buf.at[slot], sem_sim)
            lp = pltpu.async_copy(
                logp_hbm.at[pl.ds(base + c * C, C)], logp_buf.at[slot],
                sem_logp)
            return sim, lp

        smooth_acc = jnp.zeros((_L,), jnp.float32)
        ml_acc = jnp.zeros((_L,), jnp.float32)
        inflight = start_chunk(0, 0)
        inv_tau = jnp.float32(1.0 / _TAU_WORD)

        for c in range(NCHUNK):
            slot = c % 2
            inflight[0].wait()
            inflight[1].wait()
            if c + 1 < NCHUNK:
                inflight = start_chunk(c + 1, (c + 1) % 2)
            chunk_rows = c * C + rowsel
            t_vec = plsc.load_gather(idx_v, [chunk_rows])
            m_vec = plsc.load_gather(mask_v, [chunk_rows])
            picked = plsc.load_gather(
                logp_buf, [jnp.full((_L,), slot, jnp.int32), rowsel, t_vec])
            ml_acc = ml_acc + picked * m_vec * firstcopy
            for r in range(C):
                UNROLL = 8
                NACC = 4

                def body(j, carry):
                    zs = list(carry[:NACC])
                    ds_ = list(carry[NACC:])
                    base_off = j * (_L * UNROLL)
                    for u in range(UNROLL):
                        off = base_off + u * _L
                        s = sim_buf[slot, r, pl.ds(off, _L)]
                        lg = logp_buf[slot, r, pl.ds(off, _L)]
                        e = jnp.exp(s * inv_tau)
                        a = u % NACC
                        zs[a] = zs[a] + e
                        ds_[a] = ds_[a] + e * lg
                    return tuple(zs) + tuple(ds_)

                zero = jnp.zeros((_L,), jnp.float32)
                acc = lax.fori_loop(0, VCH // UNROLL, body, (zero,) * (2 * NACC))
                z = (acc[0] + acc[1]) + (acc[2] + acc[3])
                d = (acc[4] + acc[5]) + (acc[6] + acc[7])
                zt = _lane_total(z)
                dt = _lane_total(d)
                m = plsc.load_gather(
                    mask_v, [jnp.full((_L,), c * C + r, jnp.int32)])
                smooth_acc = smooth_acc + m * (dt / zt)

        msum_acc = jnp.zeros((_L,), jnp.float32)
        for g in range(RW // _L):
            msum_acc = msum_acc + mask_v[pl.ds(g * _L, _L)]

        out_stage[0, :] = smooth_acc
        out_stage[1, :] = ml_acc
        out_stage[2, :] = msum_acc
        pltpu.sync_copy(out_stage, out_hbm.at[wid])

    return sc_kernel


@jax.jit
def kernel(logp, sim_matrix, target, mask):
    B, T, V = logp.shape
    N = B * T
    flat_logp = logp.reshape(N, V)
    idx = target.reshape(N).astype(jnp.int32)
    flat_mask = mask.reshape(N)

    idx3 = idx.reshape(_NW, -1, 4)
    parts = _make_sc_kernel(N, V)(flat_logp, sim_matrix, idx, idx3,
                                  flat_mask)
    smooth_sum = jnp.sum(parts[:, 0, :]) / _L
    ml_sum = jnp.sum(parts[:, 1, :])
    msum = jnp.sum(parts[:, 2, :])
    ml_output = -ml_sum / msum
    output = -smooth_sum / msum
    final = _ALPHA * output + (1.0 - _ALPHA) * ml_output
    return jnp.stack([final, ml_output])
